# baseline (speedup 1.0000x reference)
import jax
import jax.numpy as jnp
from jax import lax
from jax.experimental import pallas as pl
from jax.experimental.pallas import tpu as pltpu
from jax.experimental.pallas import tpu_sc as plsc

EMBED_DIM = 768
B = 32
N = 4096
NUM_IDX = B * N

NUM_Q = 4
NUM_G = 8
COLS = EMBED_DIM // NUM_G
IDX_PER_Q = NUM_IDX // NUM_Q
CHUNK = 256
N_CHUNKS = IDX_PER_Q // CHUNK
GROUPS = CHUNK // 16


def _body(table_hbm, idx_hbm, out_hbm, idx_v, tab_v, stage0, stage1,
          isem, wsem0, wsem1):
    cid = lax.axis_index("core")
    sid = lax.axis_index("subcore")
    wid = sid * 2 + cid
    q = wid % NUM_Q
    g = wid // NUM_Q

    pltpu.async_copy(idx_hbm.at[pl.ds(q * IDX_PER_Q, IDX_PER_Q)], idx_v, isem)
    pltpu.sync_copy(table_hbm.at[:, pl.ds(g * COLS, COLS)], tab_v)
    pltpu.make_async_copy(
        idx_hbm.at[pl.ds(q * IDX_PER_Q, IDX_PER_Q)], idx_v, isem
    ).wait()

    stages = (stage0, stage1)
    wsems = (wsem0, wsem1)
    lanes = lax.iota(jnp.int32, 16)

    @pl.loop(0, N_CHUNKS, step=2)
    def _(c):
        for bb in range(2):
            cc = c + bb
            stage, wsem = stages[bb], wsems[bb]

            @pl.when(c >= 2)
            def _(stage=stage, wsem=wsem):
                pltpu.make_async_copy(
                    stage,
                    out_hbm.at[pl.ds(0, CHUNK), pl.ds(0, COLS)],
                    wsem,
                ).wait()

            @pl.loop(0, GROUPS)
            def _(t, cc=cc, stage=stage):
                idx_vec = idx_v[pl.ds(cc * CHUNK + t * 16, 16)]
                row_vec = lanes + t * 16
                for d in range(COLS):
                    dv = jnp.full((16,), d, jnp.int32)
                    vals = plsc.load_gather(tab_v, [idx_vec, dv])
                    plsc.store_scatter(stage, [row_vec, dv], vals)

            pltpu.async_copy(
                stage,
                out_hbm.at[pl.ds(q * IDX_PER_Q + cc * CHUNK, CHUNK),
                           pl.ds(g * COLS, COLS)],
                wsem,
            )

    for bb in range(2):
        pltpu.make_async_copy(
            stages[bb],
            out_hbm.at[pl.ds(0, CHUNK), pl.ds(0, COLS)],
            wsems[bb],
        ).wait()


def kernel(indices, spatial_embed):
    idx_flat = indices.reshape(NUM_IDX).astype(jnp.int32)
    mesh = plsc.VectorSubcoreMesh(
        core_axis_name="core", subcore_axis_name="subcore"
    )
    k = pl.kernel(
        _body,
        out_type=jax.ShapeDtypeStruct((NUM_IDX, EMBED_DIM), jnp.float32),
        mesh=mesh,
        compiler_params=pltpu.CompilerParams(
            use_tc_tiling_on_sc=False, needs_layout_passes=False
        ),
        scratch_types=[
            pltpu.VMEM((IDX_PER_Q,), jnp.int32),
            pltpu.VMEM((256, COLS), jnp.float32),
            pltpu.VMEM((CHUNK, COLS), jnp.float32),
            pltpu.VMEM((CHUNK, COLS), jnp.float32),
            pltpu.SemaphoreType.DMA,
            pltpu.SemaphoreType.DMA,
            pltpu.SemaphoreType.DMA,
        ],
    )
    out = k(spatial_embed, idx_flat)
    return out.reshape(B, N, EMBED_DIM)


# speedup vs baseline: 1.8966x; 1.8966x over previous
import jax
import jax.numpy as jnp
from jax import lax
from jax.experimental import pallas as pl
from jax.experimental.pallas import tpu as pltpu
from jax.experimental.pallas import tpu_sc as plsc

EMBED_DIM = 768
B = 32
N = 4096
NUM_IDX = B * N

NUM_Q = 4
NUM_G = 8
COLS = EMBED_DIM // NUM_G
IDX_PER_Q = NUM_IDX // NUM_Q
CHUNK = 256
N_CHUNKS = IDX_PER_Q // CHUNK
GROUPS = CHUNK // 16


def _body(table_hbm, idx_hbm, out_hbm, idx_v, tab_v, stage0, stage1,
          isem, wsem0, wsem1):
    cid = lax.axis_index("core")
    sid = lax.axis_index("subcore")
    wid = sid * 2 + cid
    q = wid % NUM_Q
    g = wid // NUM_Q

    pltpu.async_copy(idx_hbm.at[pl.ds(q * IDX_PER_Q, IDX_PER_Q)], idx_v, isem)
    pltpu.sync_copy(table_hbm.at[:, pl.ds(g * COLS, COLS)], tab_v)
    pltpu.make_async_copy(
        idx_hbm.at[pl.ds(q * IDX_PER_Q, IDX_PER_Q)], idx_v, isem
    ).wait()

    stages = (stage0, stage1)
    wsems = (wsem0, wsem1)
    lanes = lax.iota(jnp.int32, 16)

    @pl.loop(0, N_CHUNKS, step=2)
    def _(c):
        for bb in range(2):
            cc = c + bb
            stage, wsem = stages[bb], wsems[bb]

            @pl.when(c >= 2)
            def _(stage=stage, wsem=wsem):
                pltpu.make_async_copy(
                    stage,
                    out_hbm.at[pl.ds(0, CHUNK), pl.ds(0, COLS)],
                    wsem,
                ).wait()

            @pl.loop(0, GROUPS)
            def _(t, cc=cc, stage=stage):
                idx_vec = idx_v[pl.ds(cc * CHUNK + t * 16, 16)]
                row_vec = lanes + t * 16

                @plsc.parallel_loop(0, COLS, step=1, unroll=8)
                def _(d):
                    dv = jnp.full((16,), d, jnp.int32)
                    vals = plsc.load_gather(tab_v, [idx_vec, dv])
                    plsc.store_scatter(stage, [row_vec, dv], vals)

            pltpu.async_copy(
                stage,
                out_hbm.at[pl.ds(q * IDX_PER_Q + cc * CHUNK, CHUNK),
                           pl.ds(g * COLS, COLS)],
                wsem,
            )

    for bb in range(2):
        pltpu.make_async_copy(
            stages[bb],
            out_hbm.at[pl.ds(0, CHUNK), pl.ds(0, COLS)],
            wsems[bb],
        ).wait()


def kernel(indices, spatial_embed):
    idx_flat = indices.reshape(NUM_IDX).astype(jnp.int32)
    mesh = plsc.VectorSubcoreMesh(
        core_axis_name="core", subcore_axis_name="subcore"
    )
    k = pl.kernel(
        _body,
        out_type=jax.ShapeDtypeStruct((NUM_IDX, EMBED_DIM), jnp.float32),
        mesh=mesh,
        compiler_params=pltpu.CompilerParams(
            use_tc_tiling_on_sc=False, needs_layout_passes=False
        ),
        scratch_types=[
            pltpu.VMEM((IDX_PER_Q,), jnp.int32),
            pltpu.VMEM((256, COLS), jnp.float32),
            pltpu.VMEM((CHUNK, COLS), jnp.float32),
            pltpu.VMEM((CHUNK, COLS), jnp.float32),
            pltpu.SemaphoreType.DMA,
            pltpu.SemaphoreType.DMA,
            pltpu.SemaphoreType.DMA,
        ],
    )
    out = k(spatial_embed, idx_flat)
    return out.reshape(B, N, EMBED_DIM)


# speedup vs baseline: 2.1608x; 1.1393x over previous
import jax
import jax.numpy as jnp
from jax import lax
from jax.experimental import pallas as pl
from jax.experimental.pallas import tpu as pltpu
from jax.experimental.pallas import tpu_sc as plsc

EMBED_DIM = 768
B = 32
N = 4096
NUM_IDX = B * N

NUM_Q = 4
NUM_G = 8
COLS = EMBED_DIM // NUM_G
IDX_PER_Q = NUM_IDX // NUM_Q
CHUNK = 256
N_CHUNKS = IDX_PER_Q // CHUNK
GROUPS = CHUNK // 16


def _body(table_hbm, idx_hbm, out_hbm, idx_v, tab_v, stage0, stage1,
          isem, wsem0, wsem1):
    cid = lax.axis_index("core")
    sid = lax.axis_index("subcore")
    wid = sid * 2 + cid
    q = wid % NUM_Q
    g = wid // NUM_Q

    pltpu.async_copy(idx_hbm.at[pl.ds(q * IDX_PER_Q, IDX_PER_Q)], idx_v, isem)
    pltpu.sync_copy(table_hbm.at[:, pl.ds(g * COLS, COLS)], tab_v)
    pltpu.make_async_copy(
        idx_hbm.at[pl.ds(q * IDX_PER_Q, IDX_PER_Q)], idx_v, isem
    ).wait()

    stages = (stage0, stage1)
    wsems = (wsem0, wsem1)
    lanes = lax.iota(jnp.int32, 16)

    @pl.loop(0, N_CHUNKS, step=2)
    def _(c):
        for bb in range(2):
            cc = c + bb
            stage, wsem = stages[bb], wsems[bb]

            @pl.when(c >= 2)
            def _(stage=stage, wsem=wsem):
                pltpu.make_async_copy(
                    stage,
                    out_hbm.at[pl.ds(0, CHUNK), pl.ds(0, COLS)],
                    wsem,
                ).wait()

            @pl.loop(0, GROUPS)
            def _(t, cc=cc, stage=stage):
                idx_vec = idx_v[pl.ds(cc * CHUNK + t * 16, 16)]
                row_vec = lanes + t * 16
                gbase = idx_vec * COLS
                sbase = row_vec * COLS
                zeros = jnp.zeros((16,), jnp.int32)

                @plsc.parallel_loop(0, COLS, step=1, unroll=16)
                def _(d):
                    vals = plsc.load_gather(tab_v, [zeros, gbase + d])
                    plsc.store_scatter(stage, [zeros, sbase + d], vals)

            pltpu.async_copy(
                stage,
                out_hbm.at[pl.ds(q * IDX_PER_Q + cc * CHUNK, CHUNK),
                           pl.ds(g * COLS, COLS)],
                wsem,
            )

    for bb in range(2):
        pltpu.make_async_copy(
            stages[bb],
            out_hbm.at[pl.ds(0, CHUNK), pl.ds(0, COLS)],
            wsems[bb],
        ).wait()


def kernel(indices, spatial_embed):
    idx_flat = indices.reshape(NUM_IDX).astype(jnp.int32)
    mesh = plsc.VectorSubcoreMesh(
        core_axis_name="core", subcore_axis_name="subcore"
    )
    k = pl.kernel(
        _body,
        out_type=jax.ShapeDtypeStruct((NUM_IDX, EMBED_DIM), jnp.float32),
        mesh=mesh,
        compiler_params=pltpu.CompilerParams(
            use_tc_tiling_on_sc=False, needs_layout_passes=False
        ),
        scratch_types=[
            pltpu.VMEM((IDX_PER_Q,), jnp.int32),
            pltpu.VMEM((256, COLS), jnp.float32),
            pltpu.VMEM((CHUNK, COLS), jnp.float32),
            pltpu.VMEM((CHUNK, COLS), jnp.float32),
            pltpu.SemaphoreType.DMA,
            pltpu.SemaphoreType.DMA,
            pltpu.SemaphoreType.DMA,
        ],
    )
    out = k(spatial_embed, idx_flat)
    return out.reshape(B, N, EMBED_DIM)


# speedup vs baseline: 5.8249x; 2.6958x over previous
import jax
import jax.numpy as jnp
from jax import lax
from jax.experimental import pallas as pl
from jax.experimental.pallas import tpu as pltpu
from jax.experimental.pallas import tpu_sc as plsc

EMBED_DIM = 768
B = 32
N = 4096
NUM_IDX = B * N

NUM_Q = 4
NUM_G = 8
COLS = EMBED_DIM // NUM_G
IDX_PER_Q = NUM_IDX // NUM_Q
CHUNK = 256
N_CHUNKS = IDX_PER_Q // CHUNK


def _body(table_hbm, idx_hbm, out_hbm, idx_v, tab_v, stage0, stage1,
          isem, wsem0, wsem1):
    cid = lax.axis_index("core")
    sid = lax.axis_index("subcore")
    wid = sid * 2 + cid
    q = wid % NUM_Q
    g = wid // NUM_Q

    pltpu.async_copy(idx_hbm.at[pl.ds(q * IDX_PER_Q, IDX_PER_Q)], idx_v, isem)
    pltpu.sync_copy(table_hbm.at[:, pl.ds(g * COLS, COLS)], tab_v)
    pltpu.make_async_copy(
        idx_hbm.at[pl.ds(q * IDX_PER_Q, IDX_PER_Q)], idx_v, isem
    ).wait()

    stages = (stage0, stage1)
    wsems = (wsem0, wsem1)

    @pl.loop(0, N_CHUNKS, step=2)
    def _(c):
        for bb in range(2):
            cc = c + bb
            stage, wsem = stages[bb], wsems[bb]

            @pl.when(c >= 2)
            def _(stage=stage, wsem=wsem):
                pltpu.make_async_copy(
                    stage,
                    out_hbm.at[pl.ds(0, CHUNK), pl.ds(0, COLS)],
                    wsem,
                ).wait()

            @plsc.parallel_loop(0, CHUNK, step=16, unroll=2)
            def _(j, cc=cc, stage=stage):
                idx_vec = idx_v[pl.ds(cc * CHUNK + j, 16)]
                for jj in range(16):
                    sidx = idx_vec[jj]
                    for k in range(COLS // 16):
                        stage[j + jj, pl.ds(k * 16, 16)] = (
                            tab_v[sidx, pl.ds(k * 16, 16)]
                        )

            pltpu.async_copy(
                stage,
                out_hbm.at[pl.ds(q * IDX_PER_Q + cc * CHUNK, CHUNK),
                           pl.ds(g * COLS, COLS)],
                wsem,
            )

    for bb in range(2):
        pltpu.make_async_copy(
            stages[bb],
            out_hbm.at[pl.ds(0, CHUNK), pl.ds(0, COLS)],
            wsems[bb],
        ).wait()


def kernel(indices, spatial_embed):
    idx_flat = indices.reshape(NUM_IDX).astype(jnp.int32)
    mesh = plsc.VectorSubcoreMesh(
        core_axis_name="core", subcore_axis_name="subcore"
    )
    k = pl.kernel(
        _body,
        out_type=jax.ShapeDtypeStruct((NUM_IDX, EMBED_DIM), jnp.float32),
        mesh=mesh,
        compiler_params=pltpu.CompilerParams(
            use_tc_tiling_on_sc=False, needs_layout_passes=False
        ),
        scratch_types=[
            pltpu.VMEM((IDX_PER_Q,), jnp.int32),
            pltpu.VMEM((256, COLS), jnp.float32),
            pltpu.VMEM((CHUNK, COLS), jnp.float32),
            pltpu.VMEM((CHUNK, COLS), jnp.float32),
            pltpu.SemaphoreType.DMA,
            pltpu.SemaphoreType.DMA,
            pltpu.SemaphoreType.DMA,
        ],
    )
    out = k(spatial_embed, idx_flat)
    return out.reshape(B, N, EMBED_DIM)


# speedup vs baseline: 7.1601x; 1.2292x over previous
import jax
import jax.numpy as jnp
from jax import lax
from jax.experimental import pallas as pl
from jax.experimental.pallas import tpu as pltpu
from jax.experimental.pallas import tpu_sc as plsc

EMBED_DIM = 768
B = 32
N = 4096
NUM_IDX = B * N

NUM_Q = 4
NUM_G = 8
COLS = EMBED_DIM // NUM_G
IDX_PER_Q = NUM_IDX // NUM_Q
CHUNK = 256
N_CHUNKS = IDX_PER_Q // CHUNK


def _body(table_hbm, idx_hbm, out_hbm, idx_v, tab_v, stage0, stage1,
          isem, wsem0, wsem1):
    cid = lax.axis_index("core")
    sid = lax.axis_index("subcore")
    wid = sid * 2 + cid
    q = wid % NUM_Q
    g = wid // NUM_Q

    pltpu.async_copy(idx_hbm.at[pl.ds(q * IDX_PER_Q, IDX_PER_Q)],
                     idx_v.at[pl.ds(0, IDX_PER_Q)], isem)
    pltpu.sync_copy(table_hbm.at[:, pl.ds(g * COLS, COLS)], tab_v)
    pltpu.make_async_copy(
        idx_hbm.at[pl.ds(q * IDX_PER_Q, IDX_PER_Q)],
        idx_v.at[pl.ds(0, IDX_PER_Q)], isem
    ).wait()

    stages = (stage0, stage1)
    wsems = (wsem0, wsem1)

    @pl.loop(0, N_CHUNKS, step=2)
    def _(c):
        for bb in range(2):
            cc = c + bb
            stage, wsem = stages[bb], wsems[bb]

            @pl.when(c >= 2)
            def _(stage=stage, wsem=wsem):
                pltpu.make_async_copy(
                    stage,
                    out_hbm.at[pl.ds(0, CHUNK), pl.ds(0, COLS)],
                    wsem,
                ).wait()

            @plsc.parallel_loop(0, CHUNK, step=1, unroll=8)
            def _(j, cc=cc, stage=stage):
                sidx = idx_v[pl.ds(cc * CHUNK + j, 16)][0]
                vals = [tab_v[sidx, pl.ds(k * 16, 16)]
                        for k in range(COLS // 16)]
                for k in range(COLS // 16):
                    stage[j, pl.ds(k * 16, 16)] = vals[k]

            pltpu.async_copy(
                stage,
                out_hbm.at[pl.ds(q * IDX_PER_Q + cc * CHUNK, CHUNK),
                           pl.ds(g * COLS, COLS)],
                wsem,
            )

    for bb in range(2):
        pltpu.make_async_copy(
            stages[bb],
            out_hbm.at[pl.ds(0, CHUNK), pl.ds(0, COLS)],
            wsems[bb],
        ).wait()


def kernel(indices, spatial_embed):
    idx_flat = indices.reshape(NUM_IDX).astype(jnp.int32)
    mesh = plsc.VectorSubcoreMesh(
        core_axis_name="core", subcore_axis_name="subcore"
    )
    k = pl.kernel(
        _body,
        out_type=jax.ShapeDtypeStruct((NUM_IDX, EMBED_DIM), jnp.float32),
        mesh=mesh,
        compiler_params=pltpu.CompilerParams(
            use_tc_tiling_on_sc=False, needs_layout_passes=False
        ),
        scratch_types=[
            pltpu.VMEM((IDX_PER_Q + 16,), jnp.int32),
            pltpu.VMEM((256, COLS), jnp.float32),
            pltpu.VMEM((CHUNK, COLS), jnp.float32),
            pltpu.VMEM((CHUNK, COLS), jnp.float32),
            pltpu.SemaphoreType.DMA,
            pltpu.SemaphoreType.DMA,
            pltpu.SemaphoreType.DMA,
        ],
    )
    out = k(spatial_embed, idx_flat)
    return out.reshape(B, N, EMBED_DIM)


# speedup vs baseline: 7.1694x; 1.0013x over previous
import jax
import jax.numpy as jnp
from jax import lax
from jax.experimental import pallas as pl
from jax.experimental.pallas import tpu as pltpu
from jax.experimental.pallas import tpu_sc as plsc

EMBED_DIM = 768
B = 32
N = 4096
NUM_IDX = B * N

NUM_Q = 4
NUM_G = 8
COLS = EMBED_DIM // NUM_G
IDX_PER_Q = NUM_IDX // NUM_Q
CHUNK = 256
N_CHUNKS = IDX_PER_Q // CHUNK


def _body(table_hbm, idx_hbm, out_hbm, idx_v, tab_v, stage0, stage1,
          isem, wsem0, wsem1):
    cid = lax.axis_index("core")
    sid = lax.axis_index("subcore")
    wid = sid * 2 + cid
    q = wid % NUM_Q
    g = wid // NUM_Q

    pltpu.async_copy(idx_hbm.at[pl.ds(q * IDX_PER_Q, IDX_PER_Q)],
                     idx_v.at[pl.ds(0, IDX_PER_Q)], isem)
    pltpu.sync_copy(table_hbm.at[:, pl.ds(g * COLS, COLS)], tab_v)
    pltpu.make_async_copy(
        idx_hbm.at[pl.ds(q * IDX_PER_Q, IDX_PER_Q)],
        idx_v.at[pl.ds(0, IDX_PER_Q)], isem
    ).wait()

    stages = (stage0, stage1)
    wsems = (wsem0, wsem1)

    @pl.loop(0, N_CHUNKS, step=2)
    def _(c):
        for bb in range(2):
            cc = c + bb
            stage, wsem = stages[bb], wsems[bb]

            @pl.when(c >= 2)
            def _(stage=stage, wsem=wsem):
                pltpu.make_async_copy(
                    stage,
                    out_hbm.at[pl.ds(0, CHUNK), pl.ds(0, COLS)],
                    wsem,
                ).wait()

            @plsc.parallel_loop(0, CHUNK, step=1, unroll=16)
            def _(j, cc=cc, stage=stage):
                sidx = idx_v[pl.ds(cc * CHUNK + j, 16)][0]
                vals = [tab_v[sidx, pl.ds(k * 16, 16)]
                        for k in range(COLS // 16)]
                for k in range(COLS // 16):
                    stage[j, pl.ds(k * 16, 16)] = vals[k]

            pltpu.async_copy(
                stage,
                out_hbm.at[pl.ds(q * IDX_PER_Q + cc * CHUNK, CHUNK),
                           pl.ds(g * COLS, COLS)],
                wsem,
            )

    for bb in range(2):
        pltpu.make_async_copy(
            stages[bb],
            out_hbm.at[pl.ds(0, CHUNK), pl.ds(0, COLS)],
            wsems[bb],
        ).wait()


def kernel(indices, spatial_embed):
    idx_flat = indices.reshape(NUM_IDX).astype(jnp.int32)
    mesh = plsc.VectorSubcoreMesh(
        core_axis_name="core", subcore_axis_name="subcore"
    )
    k = pl.kernel(
        _body,
        out_type=jax.ShapeDtypeStruct((NUM_IDX, EMBED_DIM), jnp.float32),
        mesh=mesh,
        compiler_params=pltpu.CompilerParams(
            use_tc_tiling_on_sc=False, needs_layout_passes=False
        ),
        scratch_types=[
            pltpu.VMEM((IDX_PER_Q + 16,), jnp.int32),
            pltpu.VMEM((256, COLS), jnp.float32),
            pltpu.VMEM((CHUNK, COLS), jnp.float32),
            pltpu.VMEM((CHUNK, COLS), jnp.float32),
            pltpu.SemaphoreType.DMA,
            pltpu.SemaphoreType.DMA,
            pltpu.SemaphoreType.DMA,
        ],
    )
    out = k(spatial_embed, idx_flat)
    return out.reshape(B, N, EMBED_DIM)
